# Initial kernel scaffold; baseline (speedup 1.0000x reference)
import jax
import jax.numpy as jnp
from jax.experimental import pallas as pl


def kernel(x, edge_index, W_l, b_l, W_r, b_r):
    raise NotImplementedError("write your pallas kernel here")


# speedup vs baseline: 5.3781x; 5.3781x over previous
import functools

import jax
import jax.numpy as jnp
from jax import lax
from jax.experimental import pallas as pl
from jax.experimental.pallas import tpu as pltpu
from jax.experimental.pallas import tpu_sc as plsc

N_NODES = 10000
N_EDGES = 320000
F = 128
NC = 2
NS = 16
NW = NC * NS
EPW = N_EDGES // NW
CHUNK = 80
NCHUNK = EPW // CHUNK
NPAD = 10240
RPW = NPAD // NS
ZR = 128
CNTW = 16


def _sc_sum(x, src, dst):
    mesh = plsc.VectorSubcoreMesh(core_axis_name="c", subcore_axis_name="s")

    @functools.partial(
        pl.kernel,
        mesh=mesh,
        out_type=jax.ShapeDtypeStruct((NC * NPAD, F), jnp.float32),
        scratch_types=[
            pltpu.VMEM((CHUNK,), jnp.int32),
            pltpu.VMEM((CHUNK,), jnp.int32),
            pltpu.VMEM((CHUNK, F), jnp.float32),
            pltpu.VMEM((ZR, F), jnp.float32),
            pltpu.VMEM_SHARED((NPAD, F), jnp.float32),
            pltpu.SemaphoreType.DMA,
        ],
    )
    def agg(x_hbm, src_hbm, dst_hbm, sum_hbm,
            src_v, dst_v, rows_v, zrow_v, ssum, sem):
        cid = lax.axis_index("c")
        sid = lax.axis_index("s")
        wid = sid * NC + cid

        zero16 = jnp.zeros((16,), jnp.float32)

        def fill_zeros(i, carry):
            for j in range(F // 16):
                zrow_v[i, pl.ds(j * 16, 16)] = zero16
            return carry

        lax.fori_loop(0, ZR, fill_zeros, 0)

        r0 = sid * RPW
        for q in range(RPW // ZR):
            pltpu.sync_copy(zrow_v, ssum.at[pl.ds(r0 + q * ZR, ZR)])

        plsc.subcore_barrier()

        ebase = wid * EPW

        def body(c, carry):
            base = ebase + c * CHUNK
            pltpu.sync_copy(src_hbm.at[pl.ds(base, CHUNK)], src_v)
            pltpu.sync_copy(dst_hbm.at[pl.ds(base, CHUNK)], dst_v)
            pltpu.async_copy(x_hbm.at[src_v], rows_v, sem).wait()
            pltpu.sync_copy(rows_v, ssum.at[dst_v], add=True)
            return carry

        lax.fori_loop(0, NCHUNK, body, 0)

        plsc.subcore_barrier()

        out_r0 = cid * NPAD + r0
        pltpu.sync_copy(ssum.at[pl.ds(r0, RPW)], sum_hbm.at[pl.ds(out_r0, RPW)])

    return agg(x, src, dst)


def _sc_count(dst):
    mesh = plsc.VectorSubcoreMesh(core_axis_name="c", subcore_axis_name="s")

    @functools.partial(
        pl.kernel,
        mesh=mesh,
        out_type=jax.ShapeDtypeStruct((NC * NPAD,), jnp.float32),
        scratch_types=[
            pltpu.VMEM((CHUNK,), jnp.int32),
            pltpu.VMEM((CHUNK,), jnp.float32),
            pltpu.VMEM((RPW,), jnp.float32),
            pltpu.VMEM_SHARED((NPAD,), jnp.float32),
        ],
    )
    def agg(dst_hbm, cnt_hbm, dst_v, ones_v, zcnt_v, scnt):
        cid = lax.axis_index("c")
        sid = lax.axis_index("s")
        wid = sid * NC + cid

        zero16 = jnp.zeros((16,), jnp.float32)
        one16 = jnp.ones((16,), jnp.float32)

        def fill_zeros(i, carry):
            zcnt_v[pl.ds(i * 16, 16)] = zero16
            return carry

        lax.fori_loop(0, RPW // 16, fill_zeros, 0)

        def fill_ones(i, carry):
            ones_v[pl.ds(i * 16, 16)] = one16
            return carry

        lax.fori_loop(0, CHUNK // 16, fill_ones, 0)

        r0 = sid * RPW
        pltpu.sync_copy(zcnt_v, scnt.at[pl.ds(r0, RPW)])

        plsc.subcore_barrier()

        ebase = wid * EPW

        def body(c, carry):
            base = ebase + c * CHUNK
            pltpu.sync_copy(dst_hbm.at[pl.ds(base, CHUNK)], dst_v)
            pltpu.sync_copy(ones_v, scnt.at[dst_v], add=True)
            return carry

        lax.fori_loop(0, NCHUNK, body, 0)

        plsc.subcore_barrier()

        out_r0 = cid * NPAD + r0
        pltpu.sync_copy(scnt.at[pl.ds(r0, RPW)], cnt_hbm.at[pl.ds(out_r0, RPW)])

    return agg(dst)


def _tc_combine(x, p0, p1, c0, c1, wlT, wrT, bias):
    R = 1000

    def body(x_ref, p0_ref, p1_ref, c0_ref, c1_ref, wl_ref, wr_ref, b_ref,
             o_ref):
        s = p0_ref[...] + p1_ref[...]
        cnt = c0_ref[...] + c1_ref[...]
        mean = s / jnp.maximum(cnt, 1.0)
        acc = jnp.dot(mean, wl_ref[...], preferred_element_type=jnp.float32)
        acc = acc + jnp.dot(x_ref[...], wr_ref[...],
                            preferred_element_type=jnp.float32)
        o_ref[...] = jnp.maximum(acc + b_ref[...], 0.0)

    return pl.pallas_call(
        body,
        grid=(N_NODES // R,),
        in_specs=[
            pl.BlockSpec((R, F), lambda i: (i, 0)),
            pl.BlockSpec((R, F), lambda i: (i, 0)),
            pl.BlockSpec((R, F), lambda i: (i, 0)),
            pl.BlockSpec((R, 1), lambda i: (i, 0)),
            pl.BlockSpec((R, 1), lambda i: (i, 0)),
            pl.BlockSpec((F, F), lambda i: (0, 0)),
            pl.BlockSpec((F, F), lambda i: (0, 0)),
            pl.BlockSpec((1, F), lambda i: (0, 0)),
        ],
        out_specs=pl.BlockSpec((R, F), lambda i: (i, 0)),
        out_shape=jax.ShapeDtypeStruct((N_NODES, F), jnp.float32),
    )(x, p0, p1, c0, c1, wlT, wrT, bias)


def kernel(x, edge_index, W_l, b_l, W_r, b_r):
    ei = edge_index.astype(jnp.int32)
    src = ei[0]
    dst = ei[1]
    psum = _sc_sum(x, src, dst)
    pcnt = _sc_count(dst)
    p0 = psum[:N_NODES]
    p1 = psum[NPAD:NPAD + N_NODES]
    c0 = pcnt[:N_NODES].reshape(N_NODES, 1)
    c1 = pcnt[NPAD:NPAD + N_NODES].reshape(N_NODES, 1)
    wlT = W_l.T
    wrT = W_r.T
    bias = (b_l + b_r).reshape(1, F)
    return _tc_combine(x, p0, p1, c0, c1, wlT, wrT, bias)


# speedup vs baseline: 6.0326x; 1.1217x over previous
import functools

import jax
import jax.numpy as jnp
from jax import lax
from jax.experimental import pallas as pl
from jax.experimental.pallas import tpu as pltpu
from jax.experimental.pallas import tpu_sc as plsc

N_NODES = 10000
N_EDGES = 320000
F = 128
NC = 2
NS = 16
NW = NC * NS
EPW = N_EDGES // NW
CHUNK = 80
NCHUNK = EPW // CHUNK
NPAD = 10240
RPW = NPAD // NS
ZR = 128


def _sc_aggregate(x, src, dst):
    mesh = plsc.VectorSubcoreMesh(core_axis_name="c", subcore_axis_name="s")

    @functools.partial(
        pl.kernel,
        mesh=mesh,
        out_type=[
            jax.ShapeDtypeStruct((NC * NPAD, F), jnp.float32),
            jax.ShapeDtypeStruct((NC * NPAD,), jnp.float32),
        ],
        scratch_types=[
            pltpu.VMEM((CHUNK,), jnp.int32),
            pltpu.VMEM((CHUNK,), jnp.int32),
            pltpu.VMEM((CHUNK, F), jnp.float32),
            pltpu.VMEM((CHUNK,), jnp.float32),
            pltpu.VMEM((ZR, F), jnp.float32),
            pltpu.VMEM((RPW,), jnp.float32),
            pltpu.VMEM_SHARED((NPAD, F), jnp.float32),
            pltpu.VMEM_SHARED((NPAD,), jnp.float32),
            pltpu.SemaphoreType.DMA,
        ],
    )
    def agg(x_hbm, src_hbm, dst_hbm, sum_hbm, cnt_hbm,
            src_v, dst_v, rows_v, ones_v, zrow_v, zcnt_v, ssum, scnt, sem):
        cid = lax.axis_index("c")
        sid = lax.axis_index("s")
        wid = sid * NC + cid

        zero16 = jnp.zeros((16,), jnp.float32)
        one16 = jnp.ones((16,), jnp.float32)

        def fill_zeros(i, carry):
            for j in range(F // 16):
                zrow_v[i, pl.ds(j * 16, 16)] = zero16
            return carry

        lax.fori_loop(0, ZR, fill_zeros, 0)

        def fill_ones(i, carry):
            ones_v[pl.ds(i * 16, 16)] = one16
            return carry

        lax.fori_loop(0, CHUNK // 16, fill_ones, 0)

        def fill_zcnt(i, carry):
            zcnt_v[pl.ds(i * 16, 16)] = zero16
            return carry

        lax.fori_loop(0, RPW // 16, fill_zcnt, 0)

        r0 = sid * RPW
        for q in range(RPW // ZR):
            pltpu.sync_copy(zrow_v, ssum.at[pl.ds(r0 + q * ZR, ZR)])
        pltpu.sync_copy(zcnt_v, scnt.at[pl.ds(r0, RPW)])

        plsc.subcore_barrier()

        ebase = wid * EPW

        def body(c, carry):
            base = ebase + c * CHUNK
            pltpu.sync_copy(src_hbm.at[pl.ds(base, CHUNK)], src_v)
            pltpu.sync_copy(dst_hbm.at[pl.ds(base, CHUNK)], dst_v)
            pltpu.async_copy(x_hbm.at[src_v], rows_v, sem).wait()
            pltpu.sync_copy(rows_v, ssum.at[dst_v], add=True)
            pltpu.sync_copy(ones_v, scnt.at[dst_v], add=True)
            return carry

        lax.fori_loop(0, NCHUNK, body, 0)

        plsc.subcore_barrier()

        out_r0 = cid * NPAD + r0
        pltpu.sync_copy(ssum.at[pl.ds(r0, RPW)], sum_hbm.at[pl.ds(out_r0, RPW)])
        pltpu.sync_copy(scnt.at[pl.ds(r0, RPW)], cnt_hbm.at[pl.ds(out_r0, RPW)])

    return agg(x, src, dst)


def _tc_combine(x, p0, p1, c0, c1, wlT, wrT, bias):
    R = 1000

    def body(x_ref, p0_ref, p1_ref, c0_ref, c1_ref, wl_ref, wr_ref, b_ref,
             o_ref):
        s = p0_ref[...] + p1_ref[...]
        cnt = c0_ref[...] + c1_ref[...]
        mean = s / jnp.maximum(cnt, 1.0)
        acc = jnp.dot(mean, wl_ref[...], preferred_element_type=jnp.float32)
        acc = acc + jnp.dot(x_ref[...], wr_ref[...],
                            preferred_element_type=jnp.float32)
        o_ref[...] = jnp.maximum(acc + b_ref[...], 0.0)

    return pl.pallas_call(
        body,
        grid=(N_NODES // R,),
        in_specs=[
            pl.BlockSpec((R, F), lambda i: (i, 0)),
            pl.BlockSpec((R, F), lambda i: (i, 0)),
            pl.BlockSpec((R, F), lambda i: (i, 0)),
            pl.BlockSpec((R, 1), lambda i: (i, 0)),
            pl.BlockSpec((R, 1), lambda i: (i, 0)),
            pl.BlockSpec((F, F), lambda i: (0, 0)),
            pl.BlockSpec((F, F), lambda i: (0, 0)),
            pl.BlockSpec((1, F), lambda i: (0, 0)),
        ],
        out_specs=pl.BlockSpec((R, F), lambda i: (i, 0)),
        out_shape=jax.ShapeDtypeStruct((N_NODES, F), jnp.float32),
    )(x, p0, p1, c0, c1, wlT, wrT, bias)


def kernel(x, edge_index, W_l, b_l, W_r, b_r):
    ei = edge_index.astype(jnp.int32)
    src = ei[0]
    dst = ei[1]
    psum, pcnt = _sc_aggregate(x, src, dst)
    p0 = psum[:N_NODES]
    p1 = psum[NPAD:NPAD + N_NODES]
    c0 = pcnt[:N_NODES].reshape(N_NODES, 1)
    c1 = pcnt[NPAD:NPAD + N_NODES].reshape(N_NODES, 1)
    wlT = W_l.T
    wrT = W_r.T
    bias = (b_l + b_r).reshape(1, F)
    return _tc_combine(x, p0, p1, c0, c1, wlT, wrT, bias)


# speedup vs baseline: 10.8484x; 1.7983x over previous
import functools

import jax
import jax.numpy as jnp
from jax import lax
from jax.experimental import pallas as pl
from jax.experimental.pallas import tpu as pltpu
from jax.experimental.pallas import tpu_sc as plsc

N_NODES = 10000
N_EDGES = 320000
F = 128
NC = 2
NS = 16
NW = NC * NS
EPW = N_EDGES // NW
CHUNK = 80
NCHUNK = EPW // CHUNK
NPAD = 10240
RPW = NPAD // NS
ZR = 32


def _sc_aggregate(x, src, dst):
    mesh = plsc.VectorSubcoreMesh(core_axis_name="c", subcore_axis_name="s")

    @functools.partial(
        pl.kernel,
        mesh=mesh,
        out_type=[
            jax.ShapeDtypeStruct((NC * NPAD, F), jnp.float32),
            jax.ShapeDtypeStruct((NC * NPAD,), jnp.float32),
        ],
        scratch_types=[
            pltpu.VMEM((CHUNK,), jnp.int32),
            pltpu.VMEM((CHUNK,), jnp.int32),
            pltpu.VMEM((CHUNK,), jnp.int32),
            pltpu.VMEM((CHUNK,), jnp.int32),
            pltpu.VMEM((CHUNK, F), jnp.float32),
            pltpu.VMEM((CHUNK, F), jnp.float32),
            pltpu.VMEM((CHUNK,), jnp.float32),
            pltpu.VMEM((ZR, F), jnp.float32),
            pltpu.VMEM((RPW,), jnp.float32),
            pltpu.VMEM_SHARED((NPAD, F), jnp.float32),
            pltpu.VMEM_SHARED((NPAD,), jnp.float32),
            pltpu.SemaphoreType.DMA,
            pltpu.SemaphoreType.DMA,
            pltpu.SemaphoreType.DMA,
            pltpu.SemaphoreType.DMA,
        ],
    )
    def agg(x_hbm, src_hbm, dst_hbm, sum_hbm, cnt_hbm,
            src_a, src_b, dst_a, dst_b, rows_a, rows_b, ones_v,
            zrow_v, zcnt_v, ssum, scnt, sem_ia, sem_ib, sem_ga, sem_gb):
        cid = lax.axis_index("c")
        sid = lax.axis_index("s")
        wid = sid * NC + cid

        zero16 = jnp.zeros((16,), jnp.float32)
        one16 = jnp.ones((16,), jnp.float32)

        def fill_zeros(i, carry):
            for j in range(F // 16):
                zrow_v[i, pl.ds(j * 16, 16)] = zero16
            return carry

        lax.fori_loop(0, ZR, fill_zeros, 0)

        def fill_ones(i, carry):
            ones_v[pl.ds(i * 16, 16)] = one16
            return carry

        lax.fori_loop(0, CHUNK // 16, fill_ones, 0)

        def fill_zcnt(i, carry):
            zcnt_v[pl.ds(i * 16, 16)] = zero16
            return carry

        lax.fori_loop(0, RPW // 16, fill_zcnt, 0)

        r0 = sid * RPW
        for q in range(RPW // ZR):
            pltpu.sync_copy(zrow_v, ssum.at[pl.ds(r0 + q * ZR, ZR)])
        pltpu.sync_copy(zcnt_v, scnt.at[pl.ds(r0, RPW)])

        plsc.subcore_barrier()

        ebase = wid * EPW

        def fire_idx(c, src_buf, dst_buf, sem):
            base = ebase + c * CHUNK
            pltpu.async_copy(src_hbm.at[pl.ds(base, CHUNK)], src_buf, sem)
            pltpu.async_copy(dst_hbm.at[pl.ds(base, CHUNK)], dst_buf, sem)

        def wait_idx(src_buf, dst_buf, sem):
            pltpu.make_async_copy(src_hbm.at[pl.ds(0, CHUNK)], src_buf,
                                  sem).wait()
            pltpu.make_async_copy(dst_hbm.at[pl.ds(0, CHUNK)], dst_buf,
                                  sem).wait()

        def fire_gather(src_buf, rows_buf, sem):
            pltpu.async_copy(x_hbm.at[src_buf], rows_buf, sem)

        def wait_gather(src_buf, rows_buf, sem):
            pltpu.make_async_copy(x_hbm.at[src_buf], rows_buf, sem).wait()

        def scatter(dst_buf, rows_buf):
            pltpu.sync_copy(rows_buf, ssum.at[dst_buf], add=True)
            pltpu.sync_copy(ones_v, scnt.at[dst_buf], add=True)

        fire_idx(0, src_a, dst_a, sem_ia)
        fire_idx(1, src_b, dst_b, sem_ib)
        wait_idx(src_a, dst_a, sem_ia)
        fire_gather(src_a, rows_a, sem_ga)

        def body(k, carry):
            c = 2 * k
            wait_idx(src_b, dst_b, sem_ib)
            fire_gather(src_b, rows_b, sem_gb)
            wait_gather(src_a, rows_a, sem_ga)
            scatter(dst_a, rows_a)
            fire_idx(c + 2, src_a, dst_a, sem_ia)
            wait_idx(src_a, dst_a, sem_ia)
            fire_gather(src_a, rows_a, sem_ga)
            wait_gather(src_b, rows_b, sem_gb)
            scatter(dst_b, rows_b)
            fire_idx(c + 3, src_b, dst_b, sem_ib)
            return carry

        lax.fori_loop(0, (NCHUNK - 1) // 2 - 1, body, 0)

        c = NCHUNK - 3
        wait_idx(src_b, dst_b, sem_ib)
        fire_gather(src_b, rows_b, sem_gb)
        wait_gather(src_a, rows_a, sem_ga)
        scatter(dst_a, rows_a)
        fire_idx(c + 2, src_a, dst_a, sem_ia)
        wait_idx(src_a, dst_a, sem_ia)
        fire_gather(src_a, rows_a, sem_ga)
        wait_gather(src_b, rows_b, sem_gb)
        scatter(dst_b, rows_b)
        wait_gather(src_a, rows_a, sem_ga)
        scatter(dst_a, rows_a)

        plsc.subcore_barrier()

        out_r0 = cid * NPAD + r0
        pltpu.sync_copy(ssum.at[pl.ds(r0, RPW)], sum_hbm.at[pl.ds(out_r0, RPW)])
        pltpu.sync_copy(scnt.at[pl.ds(r0, RPW)], cnt_hbm.at[pl.ds(out_r0, RPW)])

    return agg(x, src, dst)


def _tc_combine(x, p0, p1, c0, c1, wlT, wrT, bias):
    R = 1000

    def body(x_ref, p0_ref, p1_ref, c0_ref, c1_ref, wl_ref, wr_ref, b_ref,
             o_ref):
        s = p0_ref[...] + p1_ref[...]
        cnt = c0_ref[...] + c1_ref[...]
        mean = s / jnp.maximum(cnt, 1.0)
        acc = jnp.dot(mean, wl_ref[...], preferred_element_type=jnp.float32)
        acc = acc + jnp.dot(x_ref[...], wr_ref[...],
                            preferred_element_type=jnp.float32)
        o_ref[...] = jnp.maximum(acc + b_ref[...], 0.0)

    return pl.pallas_call(
        body,
        grid=(N_NODES // R,),
        in_specs=[
            pl.BlockSpec((R, F), lambda i: (i, 0)),
            pl.BlockSpec((R, F), lambda i: (i, 0)),
            pl.BlockSpec((R, F), lambda i: (i, 0)),
            pl.BlockSpec((R, 1), lambda i: (i, 0)),
            pl.BlockSpec((R, 1), lambda i: (i, 0)),
            pl.BlockSpec((F, F), lambda i: (0, 0)),
            pl.BlockSpec((F, F), lambda i: (0, 0)),
            pl.BlockSpec((1, F), lambda i: (0, 0)),
        ],
        out_specs=pl.BlockSpec((R, F), lambda i: (i, 0)),
        out_shape=jax.ShapeDtypeStruct((N_NODES, F), jnp.float32),
    )(x, p0, p1, c0, c1, wlT, wrT, bias)


def kernel(x, edge_index, W_l, b_l, W_r, b_r):
    ei = edge_index.astype(jnp.int32)
    src = ei[0]
    dst = ei[1]
    psum, pcnt = _sc_aggregate(x, src, dst)
    p0 = psum[:N_NODES]
    p1 = psum[NPAD:NPAD + N_NODES]
    c0 = pcnt[:N_NODES].reshape(N_NODES, 1)
    c1 = pcnt[NPAD:NPAD + N_NODES].reshape(N_NODES, 1)
    wlT = W_l.T
    wrT = W_r.T
    bias = (b_l + b_r).reshape(1, F)
    return _tc_combine(x, p0, p1, c0, c1, wlT, wrT, bias)
